# Initial kernel scaffold; baseline (speedup 1.0000x reference)
import jax
import jax.numpy as jnp
from jax.experimental import pallas as pl


def kernel(detector_data):
    raise NotImplementedError("write your pallas kernel here")


# speedup vs baseline: 632.3525x; 632.3525x over previous
import functools

import jax
import jax.numpy as jnp
from jax import lax
from jax.experimental import pallas as pl
from jax.experimental.pallas import tpu as pltpu
from jax.experimental.pallas import tpu_sc as plsc

H = 4096
W = 4096
C = 16
L = 16
SW = W // C

NC = 2
NS = 16
NW = NC * NS
ROWS_PER_W = H // NW
CHUNK = 8
NCHUNK = ROWS_PER_W // CHUNK

_mesh = plsc.VectorSubcoreMesh(core_axis_name="c", subcore_axis_name="s")


def _tree_sum(vs):
    while len(vs) > 1:
        nxt = [vs[k] + vs[k + 1] for k in range(0, len(vs) - 1, 2)]
        if len(vs) % 2:
            nxt.append(vs[-1])
        vs = nxt
    return vs[0]


def _accum_chunk(buf, accs):
    def row_body(r, accs):
        out = []
        for c in range(C):
            vs = [buf[r, pl.ds(c * SW + j * L, L)] for j in range(L)]
            out.append(accs[c] + _tree_sum(vs))
        return tuple(out)

    return lax.fori_loop(0, CHUNK, row_body, accs, unroll=False)


@functools.partial(
    pl.kernel,
    out_type=jax.ShapeDtypeStruct((NW * C, L), jnp.float32),
    mesh=_mesh,
    scratch_types=[
        pltpu.VMEM((CHUNK, W), jnp.float32),
        pltpu.VMEM((CHUNK, W), jnp.float32),
        pltpu.VMEM((C, L), jnp.float32),
        pltpu.SemaphoreType.DMA,
        pltpu.SemaphoreType.DMA,
    ],
)
def _partial_sums(data_hbm, part_hbm, buf0, buf1, mat_v, sem0, sem1):
    wid = lax.axis_index("s") * NC + lax.axis_index("c")
    base = wid * ROWS_PER_W

    def chunk_slice(i):
        return data_hbm.at[pl.ds(base + i * CHUNK, CHUNK)]

    pltpu.async_copy(chunk_slice(0), buf0, sem0)
    pltpu.async_copy(chunk_slice(1), buf1, sem1)

    accs0 = tuple(jnp.zeros((L,), jnp.float32) for _ in range(C))

    def outer(i, accs):
        c0 = 2 * i
        pltpu.make_async_copy(chunk_slice(c0), buf0, sem0).wait()
        accs = _accum_chunk(buf0, accs)

        @pl.when(c0 + 2 < NCHUNK)
        def _():
            pltpu.async_copy(chunk_slice(c0 + 2), buf0, sem0)

        pltpu.make_async_copy(chunk_slice(c0 + 1), buf1, sem1).wait()
        accs = _accum_chunk(buf1, accs)

        @pl.when(c0 + 3 < NCHUNK)
        def _():
            pltpu.async_copy(chunk_slice(c0 + 3), buf1, sem1)

        return accs

    accs = lax.fori_loop(0, NCHUNK // 2, outer, accs0, unroll=False)

    for c in range(C):
        mat_v[c] = accs[c]
    pltpu.sync_copy(mat_v, part_hbm.at[pl.ds(wid * C, C)])


@functools.partial(
    pl.kernel,
    out_type=jax.ShapeDtypeStruct((1, C), jnp.float32),
    mesh=_mesh,
    scratch_types=[
        pltpu.VMEM((NW * C, L), jnp.float32),
        pltpu.VMEM((C,), jnp.float32),
    ],
)
def _finalize(part_hbm, out_hbm, all_v, ovec_v):
    wid = lax.axis_index("s") * NC + lax.axis_index("c")

    @pl.when(wid == 0)
    def _():
        pltpu.sync_copy(part_hbm, all_v)

        def wbody(w, accs):
            return tuple(accs[c] + all_v[w * C + c] for c in range(C))

        accs = lax.fori_loop(
            0, NW, wbody,
            tuple(jnp.zeros((L,), jnp.float32) for _ in range(C)),
            unroll=False)

        totals = [_tree_sum([accs[c][l] for l in range(L)]) for c in range(C)]
        lane = lax.iota(jnp.int32, L)
        class_tot = jnp.zeros((L,), jnp.float32)
        for c in range(C):
            class_tot = jnp.where(lane == c, totals[c], class_tot)
        total = _tree_sum(totals)
        ovec_v[...] = class_tot / total
        pltpu.sync_copy(ovec_v, out_hbm.at[0])


def kernel(detector_data):
    partials = _partial_sums(detector_data)
    return _finalize(partials)
